# baseline (speedup 1.0000x reference)
import functools
import math

import numpy as np
import jax
import jax.numpy as jnp
from jax import lax
from jax.experimental import pallas as pl
from jax.experimental.pallas import tpu as pltpu
import jax.experimental.pallas.tpu_sc as plsc


def _dft_mats(N: int):
    m = np.arange(N)[:, None].astype(np.float64)
    f = np.arange(1, N // 2 + 1)[None, :].astype(np.float64)
    CF = np.cos(2 * np.pi * m * f / N)
    SF = np.sin(2 * np.pi * m * f / N)
    SF[:, -1] = 0.0
    scale = np.where(f[0] == N // 2, 1.0 / N, 2.0 / N)[:, None]
    n = np.arange(N)[None, :].astype(np.float64)
    fc = np.arange(1, N // 2 + 1)[:, None].astype(np.float64)
    iC = scale * np.cos(2 * np.pi * fc * n / N)
    iS = scale * np.sin(2 * np.pi * fc * n / N)
    iS[-1, :] = 0.0
    FW = np.concatenate([CF, SF], axis=1)
    IM = np.concatenate([iC, iS], axis=0)
    return FW.astype(np.float32), IM.astype(np.float32)


def _split_bf16(x):
    h = x.astype(jnp.bfloat16)
    return h, (x - h.astype(jnp.float32)).astype(jnp.bfloat16)


def _dot3(x, mh, ml):
    xh, xl = _split_bf16(x)
    f32 = jnp.float32
    return (jnp.dot(xh, mh, preferred_element_type=f32)
            + jnp.dot(xl, mh, preferred_element_type=f32)
            + jnp.dot(xh, ml, preferred_element_type=f32))


def _tc_body(q_ref, k_ref, fwh_ref, fwl_ref, imh_ref, iml_ref,
             c_out_ref, wd_out_ref, *, topk):
    N = q_ref.shape[-1]
    H = N // 2
    q = q_ref[0, 0]
    k = k_ref[0, 0]

    qf = _dot3(q, fwh_ref[...], fwl_ref[...])
    kf = _dot3(k, fwh_ref[...], fwl_ref[...])
    qr, qi = qf[:, :H], qf[:, H:]
    kr, ki = kf[:, :H], kf[:, H:]
    rr = qr * kr + qi * ki
    ri = qi * kr - qr * ki
    dc = (jnp.sum(q, axis=-1, keepdims=True) * jnp.sum(k, axis=-1, keepdims=True)) * (1.0 / N)
    corr = _dot3(jnp.concatenate([rr, ri], axis=-1), imh_ref[...], iml_ref[...]) + dc

    fidx = lax.broadcasted_iota(jnp.int32, corr.shape, 1).astype(jnp.float32)
    work = corr
    ws, dds = [], []
    for _ in range(topk):
        mx = jnp.max(work, axis=-1, keepdims=True)
        dd = jnp.min(jnp.where(work == mx, fidx, 512.0), axis=-1, keepdims=True)
        ws.append(mx)
        dds.append(dd)
        work = jnp.where(fidx == dd, -jnp.inf, work)

    exps = [jnp.exp(w - ws[0]) for w in ws]
    denom = sum(exps)
    zero = jnp.zeros_like(ws[0])
    wd = jnp.concatenate(
        [e / denom for e in exps] + [zero, zero, zero] + dds + [zero, zero, zero],
        axis=-1)

    c_out_ref[0, 0] = corr
    wd_out_ref[0, 0] = wd


def _sc_body(v_hbm, wd_hbm, out_hbm, wd_v, v_v, v2_v, out_v, *, rows_per, cc, topk):
    wid = lax.axis_index("s") * 2 + lax.axis_index("c")
    base = wid * rows_per
    n_lanes = 16
    nchunk = 256 // n_lanes

    def block(t, _):
        r0 = base + t * cc
        pltpu.sync_copy(wd_hbm.at[pl.ds(r0, cc)], wd_v)
        pltpu.sync_copy(v_hbm.at[pl.ds(r0, cc)], v_v)

        def row(rr, _):
            for j in range(nchunk):
                chunk = v_v[rr, pl.ds(j * n_lanes, n_lanes)]
                v2_v[pl.ds(j * n_lanes, n_lanes)] = chunk
                v2_v[pl.ds(256 + j * n_lanes, n_lanes)] = chunk
            wdrow = wd_v[rr, pl.ds(0, 16)]
            w = [wdrow[i] for i in range(topk)]
            d = [wdrow[8 + i].astype(jnp.int32) for i in range(topk)]

            def chunk_out(c, _):
                acc = v2_v[pl.ds(d[0] + c * n_lanes, n_lanes)] * w[0]
                for i in range(1, topk):
                    acc = acc + v2_v[pl.ds(d[i] + c * n_lanes, n_lanes)] * w[i]
                out_v[rr, pl.ds(c * n_lanes, n_lanes)] = acc
                return 0

            lax.fori_loop(0, nchunk, chunk_out, 0, unroll=4)
            return 0

        lax.fori_loop(0, cc, row, 0)
        pltpu.sync_copy(out_v, out_hbm.at[pl.ds(r0, cc)])
        return 0

    lax.fori_loop(0, rows_per // cc, block, 0)


@jax.jit
def kernel(queries, keys, values):
    B, Hh, L, E = queries.shape
    topk = int(math.log(E))
    TL = 1024
    nl = L // TL
    FW, IM = _dft_mats(E)
    FWh = FW.astype(jnp.bfloat16)
    FWl = (FW - FWh.astype(np.float32)).astype(jnp.bfloat16)
    IMh = IM.astype(jnp.bfloat16)
    IMl = (IM - IMh.astype(np.float32)).astype(jnp.bfloat16)

    grid = (B, Hh, nl)
    mat_spec = pl.BlockSpec((E, E), lambda b, h, lt: (0, 0))
    row_spec = pl.BlockSpec((1, 1, TL, E), lambda b, h, lt: (b, h, lt, 0))
    corr, wd = pl.pallas_call(
        functools.partial(_tc_body, topk=topk),
        grid=grid,
        in_specs=[row_spec, row_spec, mat_spec, mat_spec, mat_spec, mat_spec],
        out_specs=[row_spec,
                   pl.BlockSpec((1, 1, TL, 16), lambda b, h, lt: (b, h, lt, 0))],
        out_shape=[jax.ShapeDtypeStruct((B, Hh, L, E), jnp.float32),
                   jax.ShapeDtypeStruct((B, Hh, L, 16), jnp.float32)],
    )(queries, keys, jnp.asarray(FWh), jnp.asarray(FWl),
      jnp.asarray(IMh), jnp.asarray(IMl))

    R = B * Hh * L
    rows_per = R // 32
    CC = 16
    mesh = plsc.VectorSubcoreMesh(core_axis_name="c", subcore_axis_name="s")
    sc_agg = pl.kernel(
        functools.partial(_sc_body, rows_per=rows_per, cc=CC, topk=topk),
        out_type=jax.ShapeDtypeStruct((R, E), jnp.float32),
        mesh=mesh,
        scratch_types=[
            pltpu.VMEM((CC, 16), jnp.float32),
            pltpu.VMEM((CC, E), jnp.float32),
            pltpu.VMEM((2 * E,), jnp.float32),
            pltpu.VMEM((CC, E), jnp.float32),
        ],
    )
    vagg = sc_agg(values.reshape(R, E), wd.reshape(R, 16))
    return vagg.reshape(B, Hh, L, E), jnp.transpose(corr, (0, 3, 1, 2))


# speedup vs baseline: 1.3028x; 1.3028x over previous
import functools
import math

import numpy as np
import jax
import jax.numpy as jnp
from jax import lax
from jax.experimental import pallas as pl
from jax.experimental.pallas import tpu as pltpu
import jax.experimental.pallas.tpu_sc as plsc


def _dft_mats(N: int):
    m = np.arange(N)[:, None].astype(np.float64)
    f = np.arange(1, N // 2 + 1)[None, :].astype(np.float64)
    CF = np.cos(2 * np.pi * m * f / N)
    SF = np.sin(2 * np.pi * m * f / N)
    SF[:, -1] = 0.0
    scale = np.where(f[0] == N // 2, 1.0 / N, 2.0 / N)[:, None]
    n = np.arange(N)[None, :].astype(np.float64)
    fc = np.arange(1, N // 2 + 1)[:, None].astype(np.float64)
    iC = scale * np.cos(2 * np.pi * fc * n / N)
    iS = scale * np.sin(2 * np.pi * fc * n / N)
    iS[-1, :] = 0.0
    FW = np.concatenate([CF, SF], axis=1)
    IM = np.concatenate([iC, iS], axis=0)
    return FW.astype(np.float32), IM.astype(np.float32)


def _split_bf16(x):
    h = x.astype(jnp.bfloat16)
    return h, (x - h.astype(jnp.float32)).astype(jnp.bfloat16)


def _dot3(x, mh, ml):
    xh, xl = _split_bf16(x)
    f32 = jnp.float32
    return (jnp.dot(xh, mh, preferred_element_type=f32)
            + jnp.dot(xl, mh, preferred_element_type=f32)
            + jnp.dot(xh, ml, preferred_element_type=f32))


def _tc_body(q_ref, k_ref, fwh_ref, fwl_ref, imh_ref, iml_ref,
             c_out_ref, wd_out_ref, *, topk):
    N = q_ref.shape[-1]
    H = N // 2
    q = q_ref[0, 0]
    k = k_ref[0, 0]

    qf = _dot3(q, fwh_ref[...], fwl_ref[...])
    kf = _dot3(k, fwh_ref[...], fwl_ref[...])
    qr, qi = qf[:, :H], qf[:, H:]
    kr, ki = kf[:, :H], kf[:, H:]
    rr = qr * kr + qi * ki
    ri = qi * kr - qr * ki
    dc = (jnp.sum(q, axis=-1, keepdims=True) * jnp.sum(k, axis=-1, keepdims=True)) * (1.0 / N)
    corr = _dot3(jnp.concatenate([rr, ri], axis=-1), imh_ref[...], iml_ref[...]) + dc

    fidx = lax.broadcasted_iota(jnp.int32, corr.shape, 1).astype(jnp.float32)
    work = corr
    ws, dds = [], []
    for _ in range(topk):
        mx = jnp.max(work, axis=-1, keepdims=True)
        dd = jnp.min(jnp.where(work == mx, fidx, 512.0), axis=-1, keepdims=True)
        ws.append(mx)
        dds.append(dd)
        work = jnp.where(fidx == dd, -jnp.inf, work)

    exps = [jnp.exp(w - ws[0]) for w in ws]
    denom = sum(exps)
    zero = jnp.zeros_like(ws[0])
    wd = jnp.concatenate(
        [e / denom for e in exps] + [zero, zero, zero] + dds + [zero, zero, zero],
        axis=-1)

    c_out_ref[0, 0] = corr
    wd_out_ref[0, 0] = wd


def _sc_body(v_hbm, wd_hbm, out_hbm, wd_v, v_v, v2_v, out_v, semw, semv, semo,
             *, rows_per, cc, topk):
    wid = lax.axis_index("s") * 2 + lax.axis_index("c")
    base = wid * rows_per
    n_lanes = 16
    nchunk = 256 // n_lanes
    nb = rows_per // cc
    iota = lax.iota(jnp.int32, n_lanes)

    def in_copies(t, slot):
        r0 = base + t * cc
        return (pltpu.make_async_copy(wd_hbm.at[pl.ds(r0, cc)], wd_v.at[slot],
                                      semw.at[slot]),
                pltpu.make_async_copy(v_hbm.at[pl.ds(r0, cc)], v_v.at[slot],
                                      semv.at[slot]))

    def out_copy(t, slot):
        r0 = base + t * cc
        return pltpu.make_async_copy(out_v.at[slot], out_hbm.at[pl.ds(r0, cc)],
                                     semo.at[slot])

    for t in range(2):
        for c in in_copies(t, t):
            c.start()

    def block(t, _):
        slot = jnp.bitwise_and(t, 1)
        for c in in_copies(t, slot):
            c.wait()
        @pl.when(t >= 2)
        def _():
            out_copy(t - 2, slot).wait()

        def row(rr, _):
            for j in range(nchunk):
                chunk = v_v[slot, rr, pl.ds(j * n_lanes, n_lanes)]
                v2_v[pl.ds(j * n_lanes, n_lanes)] = chunk
                v2_v[pl.ds(256 + j * n_lanes, n_lanes)] = chunk
            wdrow = wd_v[slot, rr, pl.ds(0, 16)]
            d = [wdrow[8 + i].astype(jnp.int32) for i in range(topk)]
            w = [wdrow[i] for i in range(topk)]
            for c in range(nchunk):
                acc = v2_v[pl.ds(d[0] + c * n_lanes, n_lanes)] * w[0]
                for i in range(1, topk):
                    acc = acc + v2_v[pl.ds(d[i] + c * n_lanes, n_lanes)] * w[i]
                out_v[slot, rr, pl.ds(c * n_lanes, n_lanes)] = acc
            return 0

        lax.fori_loop(0, cc, row, 0)
        out_copy(t, slot).start()

        @pl.when(t + 2 < nb)
        def _():
            for c in in_copies(t + 2, slot):
                c.start()
        return 0

    lax.fori_loop(0, nb, block, 0)
    out_copy(nb - 2, jnp.bitwise_and(nb - 2, 1)).wait()
    out_copy(nb - 1, jnp.bitwise_and(nb - 1, 1)).wait()


@jax.jit
def kernel(queries, keys, values):
    B, Hh, L, E = queries.shape
    topk = int(math.log(E))
    TL = 1024
    nl = L // TL
    FW, IM = _dft_mats(E)
    FWh = FW.astype(jnp.bfloat16)
    FWl = (FW - FWh.astype(np.float32)).astype(jnp.bfloat16)
    IMh = IM.astype(jnp.bfloat16)
    IMl = (IM - IMh.astype(np.float32)).astype(jnp.bfloat16)

    grid = (B, Hh, nl)
    mat_spec = pl.BlockSpec((E, E), lambda b, h, lt: (0, 0))
    row_spec = pl.BlockSpec((1, 1, TL, E), lambda b, h, lt: (b, h, lt, 0))
    corr, wd = pl.pallas_call(
        functools.partial(_tc_body, topk=topk),
        grid=grid,
        in_specs=[row_spec, row_spec, mat_spec, mat_spec, mat_spec, mat_spec],
        out_specs=[row_spec,
                   pl.BlockSpec((1, 1, TL, 16), lambda b, h, lt: (b, h, lt, 0))],
        out_shape=[jax.ShapeDtypeStruct((B, Hh, L, E), jnp.float32),
                   jax.ShapeDtypeStruct((B, Hh, L, 16), jnp.float32)],
    )(queries, keys, jnp.asarray(FWh), jnp.asarray(FWl),
      jnp.asarray(IMh), jnp.asarray(IMl))

    R = B * Hh * L
    rows_per = R // 32
    CC = 32
    mesh = plsc.VectorSubcoreMesh(core_axis_name="c", subcore_axis_name="s")
    sc_agg = pl.kernel(
        functools.partial(_sc_body, rows_per=rows_per, cc=CC, topk=topk),
        out_type=jax.ShapeDtypeStruct((R, E), jnp.float32),
        mesh=mesh,
        scratch_types=[
            pltpu.VMEM((2, CC, 16), jnp.float32),
            pltpu.VMEM((2, CC, E), jnp.float32),
            pltpu.VMEM((2 * E,), jnp.float32),
            pltpu.VMEM((2, CC, E), jnp.float32),
            pltpu.SemaphoreType.DMA((2,)),
            pltpu.SemaphoreType.DMA((2,)),
            pltpu.SemaphoreType.DMA((2,)),
        ],
    )
    vagg = sc_agg(values.reshape(R, E), wd.reshape(R, 16))
    return vagg.reshape(B, Hh, L, E), jnp.transpose(corr, (0, 3, 1, 2))


# speedup vs baseline: 3.3335x; 2.5587x over previous
import functools
import math

import numpy as np
import jax
import jax.numpy as jnp
from jax.experimental import pallas as pl


def _dft_mats(N: int):
    m = np.arange(N)[:, None].astype(np.float64)
    f = np.arange(1, N // 2 + 1)[None, :].astype(np.float64)
    CF = np.cos(2 * np.pi * m * f / N)
    SF = np.sin(2 * np.pi * m * f / N)
    SF[:, -1] = 0.0
    scale = np.where(f[0] == N // 2, 1.0 / N, 2.0 / N)[:, None]
    n = np.arange(N)[None, :].astype(np.float64)
    fc = np.arange(1, N // 2 + 1)[:, None].astype(np.float64)
    iC = scale * np.cos(2 * np.pi * fc * n / N)
    iS = scale * np.sin(2 * np.pi * fc * n / N)
    iS[-1, :] = 0.0
    FW = np.concatenate([CF, SF], axis=1)
    IM = np.concatenate([iC, iS], axis=0)
    return FW.astype(np.float32), IM.astype(np.float32)


def _split_bf16(x):
    h = x.astype(jnp.bfloat16)
    return h, (x - h.astype(jnp.float32)).astype(jnp.bfloat16)


def _dot3(x, mh, ml):
    xh, xl = _split_bf16(x)
    f32 = jnp.float32
    return (jnp.dot(xh, mh, preferred_element_type=f32)
            + jnp.dot(xl, mh, preferred_element_type=f32)
            + jnp.dot(xh, ml, preferred_element_type=f32))


def _body(q_ref, k_ref, v_ref, fw_ref, im_ref, fwh_ref, fwl_ref, imh_ref, iml_ref,
          v_out_ref, c_out_ref, *, topk):
    N = q_ref.shape[-1]
    H = N // 2
    q = q_ref[0, 0]
    k = k_ref[0, 0]
    v = v_ref[0, 0]
    fw = fw_ref[...]
    im = im_ref[...]

    qf = _dot3(q, fwh_ref[...], fwl_ref[...])
    kf = _dot3(k, fwh_ref[...], fwl_ref[...])
    qr, qi = qf[:, :H], qf[:, H:]
    kr, ki = kf[:, :H], kf[:, H:]
    rr = qr * kr + qi * ki
    ri = qi * kr - qr * ki
    dc = (jnp.sum(q, axis=-1, keepdims=True) * jnp.sum(k, axis=-1, keepdims=True)) * (1.0 / N)
    corr = _dot3(jnp.concatenate([rr, ri], axis=-1), imh_ref[...], iml_ref[...]) + dc

    fidx = jax.lax.broadcasted_iota(jnp.int32, corr.shape, 1).astype(jnp.float32)
    work = corr
    ws = []
    for _ in range(topk):
        mx = jnp.max(work, axis=-1, keepdims=True)
        dd = jnp.min(jnp.where(work == mx, fidx, 512.0), axis=-1, keepdims=True)
        ws.append(mx)
        work = jnp.where(fidx == dd, -jnp.inf, work)

    denom = sum(jnp.exp(w - ws[0]) for w in ws)
    oh = jnp.where(work == -jnp.inf, jnp.exp(corr - ws[0]) / denom, 0.0)

    vf = jnp.dot(v, fw, preferred_element_type=jnp.float32)
    of = jnp.dot(oh, fw, preferred_element_type=jnp.float32)
    vr, vi = vf[:, :H], vf[:, H:]
    orr, oi = of[:, :H], of[:, H:]
    ar = vr * orr + vi * oi
    ai = vi * orr - vr * oi
    vdc = jnp.sum(v, axis=-1, keepdims=True) * (1.0 / N)
    vagg = jnp.dot(jnp.concatenate([ar, ai], axis=-1), im,
                   preferred_element_type=jnp.float32) + vdc

    v_out_ref[0, 0] = vagg
    c_out_ref[0, 0] = corr


@jax.jit
def kernel(queries, keys, values):
    B, Hh, L, E = queries.shape
    topk = int(math.log(E))
    TL = 1024
    nl = L // TL
    FW, IM = _dft_mats(E)
    fw = jnp.asarray(FW)
    im = jnp.asarray(IM)
    FWh = FW.astype(jnp.bfloat16)
    FWl = (FW - FWh.astype(np.float32)).astype(jnp.bfloat16)
    IMh = IM.astype(jnp.bfloat16)
    IMl = (IM - IMh.astype(np.float32)).astype(jnp.bfloat16)

    grid = (B, Hh, nl)
    mat_spec = pl.BlockSpec((E, E), lambda b, h, lt: (0, 0))
    in_specs = [
        pl.BlockSpec((1, 1, TL, E), lambda b, h, lt: (b, h, lt, 0)),
        pl.BlockSpec((1, 1, TL, E), lambda b, h, lt: (b, h, lt, 0)),
        pl.BlockSpec((1, 1, TL, E), lambda b, h, lt: (b, h, lt, 0)),
        mat_spec, mat_spec, mat_spec, mat_spec, mat_spec, mat_spec,
    ]
    out_specs = [
        pl.BlockSpec((1, 1, TL, E), lambda b, h, lt: (b, h, lt, 0)),
        pl.BlockSpec((1, 1, TL, E), lambda b, h, lt: (b, h, lt, 0)),
    ]
    out_shapes = [
        jax.ShapeDtypeStruct((B, Hh, L, E), jnp.float32),
        jax.ShapeDtypeStruct((B, Hh, L, E), jnp.float32),
    ]
    vagg, corr_m = pl.pallas_call(
        functools.partial(_body, topk=topk),
        grid=grid,
        in_specs=in_specs,
        out_specs=out_specs,
        out_shape=out_shapes,
    )(queries, keys, values, fw, im,
      jnp.asarray(FWh), jnp.asarray(FWl), jnp.asarray(IMh), jnp.asarray(IMl))
    return vagg, jnp.transpose(corr_m, (0, 3, 1, 2))


# speedup vs baseline: 3.5156x; 1.0546x over previous
import functools
import math

import numpy as np
import jax
import jax.numpy as jnp
from jax.experimental import pallas as pl


def _dft_mats(N: int):
    m = np.arange(N)[:, None].astype(np.float64)
    f = np.arange(1, N // 2 + 1)[None, :].astype(np.float64)
    CF = np.cos(2 * np.pi * m * f / N)
    SF = np.sin(2 * np.pi * m * f / N)
    SF[:, -1] = 0.0
    scale = np.where(f[0] == N // 2, 1.0 / N, 2.0 / N)[:, None]
    n = np.arange(N)[None, :].astype(np.float64)
    fc = np.arange(1, N // 2 + 1)[:, None].astype(np.float64)
    iC = scale * np.cos(2 * np.pi * fc * n / N)
    iS = scale * np.sin(2 * np.pi * fc * n / N)
    iS[-1, :] = 0.0
    FW = np.concatenate([CF, SF], axis=1)
    IM = np.concatenate([iC, iS], axis=0)
    return FW.astype(np.float32), IM.astype(np.float32)


def _split_bf16(x):
    h = x.astype(jnp.bfloat16)
    return h, (x - h.astype(jnp.float32)).astype(jnp.bfloat16)


def _dot3(x, mh, ml):
    xh, xl = _split_bf16(x)
    f32 = jnp.float32
    return (jnp.dot(xh, mh, preferred_element_type=f32)
            + jnp.dot(xl, mh, preferred_element_type=f32)
            + jnp.dot(xh, ml, preferred_element_type=f32))


def _body(q_ref, k_ref, v_ref, fw_ref, im_ref, fwh_ref, fwl_ref, imh_ref, iml_ref,
          v_out_ref, c_out_ref, *, topk):
    N = q_ref.shape[-1]
    H = N // 2
    q = q_ref[0, 0]
    k = k_ref[0, 0]
    v = v_ref[0, 0]
    fw = fw_ref[...]
    im = im_ref[...]

    qf = _dot3(q, fwh_ref[...], fwl_ref[...])
    kf = _dot3(k, fwh_ref[...], fwl_ref[...])
    qr, qi = qf[:, :H], qf[:, H:]
    kr, ki = kf[:, :H], kf[:, H:]
    rr = qr * kr + qi * ki
    ri = qi * kr - qr * ki
    dc = (jnp.sum(q, axis=-1, keepdims=True) * jnp.sum(k, axis=-1, keepdims=True)) * (1.0 / N)
    corr = _dot3(jnp.concatenate([rr, ri], axis=-1), imh_ref[...], iml_ref[...]) + dc

    fidx = jax.lax.broadcasted_iota(jnp.int32, corr.shape, 1).astype(jnp.float32)
    work = corr
    ws = []
    for _ in range(topk):
        mx = jnp.max(work, axis=-1, keepdims=True)
        dd = jnp.min(jnp.where(work == mx, fidx, 512.0), axis=-1, keepdims=True)
        ws.append(mx)
        work = jnp.where(fidx == dd, -jnp.inf, work)

    denom = sum(jnp.exp(w - ws[0]) for w in ws)
    oh = jnp.where(work == -jnp.inf, jnp.exp(corr - ws[0]) / denom, 0.0)

    vf = jnp.dot(v, fw, preferred_element_type=jnp.float32)
    of = jnp.dot(oh, fw, preferred_element_type=jnp.float32)
    vr, vi = vf[:, :H], vf[:, H:]
    orr, oi = of[:, :H], of[:, H:]
    ar = vr * orr + vi * oi
    ai = vi * orr - vr * oi
    vdc = jnp.sum(v, axis=-1, keepdims=True) * (1.0 / N)
    vagg = jnp.dot(jnp.concatenate([ar, ai], axis=-1), im,
                   preferred_element_type=jnp.float32) + vdc

    v_out_ref[0, 0] = vagg
    c_out_ref[0, 0] = corr


@jax.jit
def kernel(queries, keys, values):
    B, Hh, L, E = queries.shape
    topk = int(math.log(E))
    TL = 2048
    nl = L // TL
    FW, IM = _dft_mats(E)
    fw = jnp.asarray(FW)
    im = jnp.asarray(IM)
    FWh = FW.astype(jnp.bfloat16)
    FWl = (FW - FWh.astype(np.float32)).astype(jnp.bfloat16)
    IMh = IM.astype(jnp.bfloat16)
    IMl = (IM - IMh.astype(np.float32)).astype(jnp.bfloat16)

    grid = (B, Hh, nl)
    mat_spec = pl.BlockSpec((E, E), lambda b, h, lt: (0, 0))
    in_specs = [
        pl.BlockSpec((1, 1, TL, E), lambda b, h, lt: (b, h, lt, 0)),
        pl.BlockSpec((1, 1, TL, E), lambda b, h, lt: (b, h, lt, 0)),
        pl.BlockSpec((1, 1, TL, E), lambda b, h, lt: (b, h, lt, 0)),
        mat_spec, mat_spec, mat_spec, mat_spec, mat_spec, mat_spec,
    ]
    out_specs = [
        pl.BlockSpec((1, 1, TL, E), lambda b, h, lt: (b, h, lt, 0)),
        pl.BlockSpec((1, 1, TL, E), lambda b, h, lt: (b, h, lt, 0)),
    ]
    out_shapes = [
        jax.ShapeDtypeStruct((B, Hh, L, E), jnp.float32),
        jax.ShapeDtypeStruct((B, Hh, L, E), jnp.float32),
    ]
    vagg, corr_m = pl.pallas_call(
        functools.partial(_body, topk=topk),
        grid=grid,
        in_specs=in_specs,
        out_specs=out_specs,
        out_shape=out_shapes,
    )(queries, keys, values, fw, im,
      jnp.asarray(FWh), jnp.asarray(FWl), jnp.asarray(IMh), jnp.asarray(IMl))
    return vagg, jnp.transpose(corr_m, (0, 3, 1, 2))


# speedup vs baseline: 3.6223x; 1.0303x over previous
import functools
import math

import numpy as np
import jax
import jax.numpy as jnp
from jax.experimental import pallas as pl


def _dft_mats(N: int):
    m = np.arange(N)[:, None].astype(np.float64)
    f = np.arange(1, N // 2 + 1)[None, :].astype(np.float64)
    CF = np.cos(2 * np.pi * m * f / N)
    SF = np.sin(2 * np.pi * m * f / N)
    SF[:, -1] = 0.0
    scale = np.where(f[0] == N // 2, 1.0 / N, 2.0 / N)[:, None]
    n = np.arange(N)[None, :].astype(np.float64)
    fc = np.arange(1, N // 2 + 1)[:, None].astype(np.float64)
    iC = scale * np.cos(2 * np.pi * fc * n / N)
    iS = scale * np.sin(2 * np.pi * fc * n / N)
    iS[-1, :] = 0.0
    FW = np.concatenate([CF, SF], axis=1)
    IM = np.concatenate([iC, iS], axis=0)
    return FW.astype(np.float32), IM.astype(np.float32)


def _split_bf16(x):
    h = x.astype(jnp.bfloat16)
    return h, (x - h.astype(jnp.float32)).astype(jnp.bfloat16)


def _dot3(x, mh, ml):
    xh, xl = _split_bf16(x)
    f32 = jnp.float32
    return (jnp.dot(xh, mh, preferred_element_type=f32)
            + jnp.dot(xl, mh, preferred_element_type=f32)
            + jnp.dot(xh, ml, preferred_element_type=f32))


def _body(q_ref, k_ref, v_ref, fw_ref, im_ref, fwh_ref, fwl_ref, imh_ref, iml_ref,
          v_out_ref, c_out_ref, *, topk):
    N = q_ref.shape[-1]
    H = N // 2
    q = q_ref[0, 0]
    k = k_ref[0, 0]
    v = v_ref[0, 0]
    fw = fw_ref[...]
    im = im_ref[...]

    qf = _dot3(q, fwh_ref[...], fwl_ref[...])
    kf = _dot3(k, fwh_ref[...], fwl_ref[...])
    qr, qi = qf[:, :H], qf[:, H:]
    kr, ki = kf[:, :H], kf[:, H:]
    rr = qr * kr + qi * ki
    ri = qi * kr - qr * ki
    dc = (jnp.sum(q, axis=-1, keepdims=True) * jnp.sum(k, axis=-1, keepdims=True)) * (1.0 / N)
    corr = _dot3(jnp.concatenate([rr, ri], axis=-1), imh_ref[...], iml_ref[...]) + dc

    fidx = jax.lax.broadcasted_iota(jnp.int32, corr.shape, 1).astype(jnp.float32)
    work = corr
    ws = []
    for _ in range(topk):
        mx = jnp.max(work, axis=-1, keepdims=True)
        dd = jnp.min(jnp.where(work == mx, fidx, 512.0), axis=-1, keepdims=True)
        ws.append(mx)
        work = jnp.where(fidx == dd, -jnp.inf, work)

    denom = sum(jnp.exp(w - ws[0]) for w in ws)
    oh = jnp.where(work == -jnp.inf, jnp.exp(corr - ws[0]) / denom, 0.0)

    vf = jnp.dot(v, fw, preferred_element_type=jnp.float32)
    of = jnp.dot(oh, fw, preferred_element_type=jnp.float32)
    vr, vi = vf[:, :H], vf[:, H:]
    orr, oi = of[:, :H], of[:, H:]
    ar = vr * orr + vi * oi
    ai = vi * orr - vr * oi
    vdc = jnp.sum(v, axis=-1, keepdims=True) * (1.0 / N)
    vagg = jnp.dot(jnp.concatenate([ar, ai], axis=-1), im,
                   preferred_element_type=jnp.float32) + vdc

    v_out_ref[0, 0] = vagg
    c_out_ref[0, 0] = corr


@jax.jit
def kernel(queries, keys, values):
    B, Hh, L, E = queries.shape
    topk = int(math.log(E))
    TL = 4096
    nl = L // TL
    FW, IM = _dft_mats(E)
    fw = jnp.asarray(FW)
    im = jnp.asarray(IM)
    FWh = FW.astype(jnp.bfloat16)
    FWl = (FW - FWh.astype(np.float32)).astype(jnp.bfloat16)
    IMh = IM.astype(jnp.bfloat16)
    IMl = (IM - IMh.astype(np.float32)).astype(jnp.bfloat16)

    grid = (B, Hh, nl)
    mat_spec = pl.BlockSpec((E, E), lambda b, h, lt: (0, 0))
    in_specs = [
        pl.BlockSpec((1, 1, TL, E), lambda b, h, lt: (b, h, lt, 0)),
        pl.BlockSpec((1, 1, TL, E), lambda b, h, lt: (b, h, lt, 0)),
        pl.BlockSpec((1, 1, TL, E), lambda b, h, lt: (b, h, lt, 0)),
        mat_spec, mat_spec, mat_spec, mat_spec, mat_spec, mat_spec,
    ]
    out_specs = [
        pl.BlockSpec((1, 1, TL, E), lambda b, h, lt: (b, h, lt, 0)),
        pl.BlockSpec((1, 1, TL, E), lambda b, h, lt: (b, h, lt, 0)),
    ]
    out_shapes = [
        jax.ShapeDtypeStruct((B, Hh, L, E), jnp.float32),
        jax.ShapeDtypeStruct((B, Hh, L, E), jnp.float32),
    ]
    vagg, corr_m = pl.pallas_call(
        functools.partial(_body, topk=topk),
        grid=grid,
        in_specs=in_specs,
        out_specs=out_specs,
        out_shape=out_shapes,
    )(queries, keys, values, fw, im,
      jnp.asarray(FWh), jnp.asarray(FWl), jnp.asarray(IMh), jnp.asarray(IMl))
    return vagg, jnp.transpose(corr_m, (0, 3, 1, 2))


# speedup vs baseline: 3.6961x; 1.0204x over previous
import functools
import math

import numpy as np
import jax
import jax.numpy as jnp
from jax.experimental import pallas as pl


def _dft_mats(N: int):
    m = np.arange(N)[:, None].astype(np.float64)
    f = np.arange(1, N // 2 + 1)[None, :].astype(np.float64)
    CF = np.cos(2 * np.pi * m * f / N)
    SF = np.sin(2 * np.pi * m * f / N)
    SF[:, -1] = 0.0
    scale = np.where(f[0] == N // 2, 1.0 / N, 2.0 / N)[:, None]
    n = np.arange(N)[None, :].astype(np.float64)
    fc = np.arange(1, N // 2 + 1)[:, None].astype(np.float64)
    iC = scale * np.cos(2 * np.pi * fc * n / N)
    iS = scale * np.sin(2 * np.pi * fc * n / N)
    iS[-1, :] = 0.0
    FW = np.concatenate([CF, SF], axis=1)
    IM = np.concatenate([iC, iS], axis=0)
    return FW.astype(np.float32), IM.astype(np.float32)


def _split_bf16(x):
    h = x.astype(jnp.bfloat16)
    return h, (x - h.astype(jnp.float32)).astype(jnp.bfloat16)


def _dot3(x, mh, ml):
    xh, xl = _split_bf16(x)
    f32 = jnp.float32
    return (jnp.dot(xh, mh, preferred_element_type=f32)
            + jnp.dot(xl, mh, preferred_element_type=f32)
            + jnp.dot(xh, ml, preferred_element_type=f32))


def _body(q_ref, k_ref, v_ref, fw_ref, im_ref, fwh_ref, fwl_ref, imh_ref, iml_ref,
          v_out_ref, c_out_ref, *, topk):
    N = q_ref.shape[-1]
    H = N // 2
    q = q_ref[0, 0]
    k = k_ref[0, 0]
    v = v_ref[0, 0]
    fw = fw_ref[...]
    im = im_ref[...]

    qf = _dot3(q, fwh_ref[...], fwl_ref[...])
    kf = _dot3(k, fwh_ref[...], fwl_ref[...])
    qr, qi = qf[:, :H], qf[:, H:]
    kr, ki = kf[:, :H], kf[:, H:]
    rr = qr * kr + qi * ki
    ri = qi * kr - qr * ki
    dc = (jnp.sum(q, axis=-1, keepdims=True) * jnp.sum(k, axis=-1, keepdims=True)) * (1.0 / N)
    corr = _dot3(jnp.concatenate([rr, ri], axis=-1), imh_ref[...], iml_ref[...]) + dc

    fidx = jax.lax.broadcasted_iota(jnp.int32, corr.shape, 1).astype(jnp.float32)
    work = corr
    ws = []
    for _ in range(topk):
        mx = jnp.max(work, axis=-1, keepdims=True)
        dd = jnp.min(jnp.where(work == mx, fidx, 512.0), axis=-1, keepdims=True)
        ws.append(mx)
        work = jnp.where(fidx == dd, -jnp.inf, work)

    e = jnp.where(work == -jnp.inf, jnp.exp(corr - ws[0]), 0.0)
    denom = jnp.sum(e, axis=-1, keepdims=True)
    oh = e * (1.0 / denom)

    vf = jnp.dot(v, fw, preferred_element_type=jnp.float32)
    of = jnp.dot(oh, fw, preferred_element_type=jnp.float32)
    vr, vi = vf[:, :H], vf[:, H:]
    orr, oi = of[:, :H], of[:, H:]
    ar = vr * orr + vi * oi
    ai = vi * orr - vr * oi
    vdc = jnp.sum(v, axis=-1, keepdims=True) * (1.0 / N)
    vagg = jnp.dot(jnp.concatenate([ar, ai], axis=-1), im,
                   preferred_element_type=jnp.float32) + vdc

    v_out_ref[0, 0] = vagg
    c_out_ref[0] = corr.T


@jax.jit
def kernel(queries, keys, values):
    B, Hh, L, E = queries.shape
    topk = int(math.log(E))
    TL = 4096
    nl = L // TL
    FW, IM = _dft_mats(E)
    fw = jnp.asarray(FW)
    im = jnp.asarray(IM)
    FWh = FW.astype(jnp.bfloat16)
    FWl = (FW - FWh.astype(np.float32)).astype(jnp.bfloat16)
    IMh = IM.astype(jnp.bfloat16)
    IMl = (IM - IMh.astype(np.float32)).astype(jnp.bfloat16)

    grid = (B, Hh, nl)
    mat_spec = pl.BlockSpec((E, E), lambda b, h, lt: (0, 0))
    in_specs = [
        pl.BlockSpec((1, 1, TL, E), lambda b, h, lt: (b, h, lt, 0)),
        pl.BlockSpec((1, 1, TL, E), lambda b, h, lt: (b, h, lt, 0)),
        pl.BlockSpec((1, 1, TL, E), lambda b, h, lt: (b, h, lt, 0)),
        mat_spec, mat_spec, mat_spec, mat_spec, mat_spec, mat_spec,
    ]
    out_specs = [
        pl.BlockSpec((1, 1, TL, E), lambda b, h, lt: (b, h, lt, 0)),
        pl.BlockSpec((1, E, TL), lambda b, h, lt: (b, 0, h * nl + lt)),
    ]
    out_shapes = [
        jax.ShapeDtypeStruct((B, Hh, L, E), jnp.float32),
        jax.ShapeDtypeStruct((B, E, Hh * L), jnp.float32),
    ]
    vagg, corr_m = pl.pallas_call(
        functools.partial(_body, topk=topk),
        grid=grid,
        in_specs=in_specs,
        out_specs=out_specs,
        out_shape=out_shapes,
    )(queries, keys, values, fw, im,
      jnp.asarray(FWh), jnp.asarray(FWl), jnp.asarray(IMh), jnp.asarray(IMl))
    return vagg, corr_m.reshape(B, E, Hh, L)
